# baseline (speedup 1.0000x reference)
import functools

import jax
import jax.numpy as jnp
from jax import lax
from jax.experimental import pallas as pl
from jax.experimental.pallas import tpu as pltpu
from jax.experimental.pallas import tpu_sc as plsc

ROWS, COLS = 2 * 8192, 2048
NC, NS, L = 2, 16, 16
NW = NC * NS

H_TC = 10240
H_SC = ROWS - H_TC
RW = H_SC // NW
SLABS = RW // 32
FCH = 4
NCH = SLABS * FCH
CE = 8 * COLS
ME = 32 * COLS
QG = CE // 64

BT = 512


def _tc_body(out_ref, tgt_ref, msk_ref, sums_ref, cnts_ref, acc, cnt):
    @pl.when(pl.program_id(0) == 0)
    def _():
        acc[...] = jnp.zeros_like(acc)
        cnt[...] = jnp.zeros_like(cnt)

    ones = jnp.ones((COLS,), jnp.float32)
    a = jnp.zeros((8, COLS), jnp.float32)
    c = jnp.zeros((8,), jnp.float32)
    for k in range(BT // 8):
        o = out_ref[pl.ds(8 * k, 8), :]
        t = tgt_ref[pl.ds(8 * k, 8), :]
        m = msk_ref[pl.ds(8 * k, 8), :]
        mf = m.astype(jnp.float32)
        diff = o - t
        a = a + (diff * diff) * mf
        c = c + jnp.dot(mf, ones, preferred_element_type=jnp.float32)
    acc[...] += a
    cnt[...] += c[:, None]

    @pl.when(pl.program_id(0) == pl.num_programs(0) - 1)
    def _():
        sums_ref[...] = acc[...]
        cnts_ref[...] = cnt[...]


def _tc_partial(out2d, tgt2d, msk2d):
    grid = H_TC // BT
    blk = lambda i: (i, 0)
    return pl.pallas_call(
        _tc_body,
        grid=(grid,),
        in_specs=[
            pl.BlockSpec((BT, COLS), blk),
            pl.BlockSpec((BT, COLS), blk),
            pl.BlockSpec((BT, COLS), blk),
        ],
        out_specs=[
            pl.BlockSpec((8, COLS), lambda i: (0, 0)),
            pl.BlockSpec((8, 1), lambda i: (0, 0)),
        ],
        out_shape=[
            jax.ShapeDtypeStruct((8, COLS), jnp.float32),
            jax.ShapeDtypeStruct((8, 1), jnp.float32),
        ],
        scratch_shapes=[
            pltpu.VMEM((8, COLS), jnp.float32),
            pltpu.VMEM((8, 1), jnp.float32),
        ],
        compiler_params=pltpu.CompilerParams(
            dimension_semantics=("arbitrary",)),
    )(out2d, tgt2d, msk2d)


def _sc_body(out_hbm, tgt_hbm, msk_hbm, sums_hbm, cnts_hbm,
             out_v0, out_v1, tgt_v0, tgt_v1, msk_v0, msk_v1,
             st_f, st_c, sems, msems):
    out_v = (out_v0, out_v1)
    tgt_v = (tgt_v0, tgt_v1)
    msk_v = (msk_v0, msk_v1)
    wid = lax.axis_index("s") * NC + lax.axis_index("c")
    row0 = H_TC + wid * RW

    ii = lax.iota(jnp.int32, 16)
    widx = lax.shift_right_logical(ii, 2)
    shifts = lax.shift_left(jnp.bitwise_and(ii, 3), 3)
    perms = [widx + 4 * b for b in range(4)]

    def lane_permute(x, idx):
        return lax.gather(
            x, idx[:, None],
            dimension_numbers=lax.GatherDimensionNumbers(
                offset_dims=(), collapsed_slice_dims=(0,),
                start_index_map=(0,)),
            slice_sizes=(1,),
            mode=lax.GatherScatterMode.PROMISE_IN_BOUNDS)

    def f32_copies(k, slot):
        r = row0 + (k // FCH) * 32 + (k % FCH) * 8
        return (
            pltpu.make_async_copy(out_hbm.at[pl.ds(r, 8), :],
                                  out_v[slot],
                                  sems.at[slot]),
            pltpu.make_async_copy(tgt_hbm.at[pl.ds(r, 8), :],
                                  tgt_v[slot],
                                  sems.at[slot]),
        )

    def msk_copy(s, slot):
        r = row0 + s * 32
        return pltpu.make_async_copy(msk_hbm.at[pl.ds(r, 32), :],
                                     msk_v[slot],
                                     msems.at[slot])

    def compute_chunk(fs, j, acc, cnt, ms):

        def group_body(q, carry):
            acc2, cnt2 = carry
            rr = lax.shift_right_logical(q, 5)
            cq = jnp.bitwise_and(q, 31) * 64
            w = plsc.bitcast(
                msk_v[ms][8 * j + rr, pl.ds(cq, 64)], jnp.int32)
            for b in range(4):
                d = out_v[fs][rr, pl.ds(cq + b * 16, 16)]
                e = tgt_v[fs][rr, pl.ds(cq + b * 16, 16)]
                wb = lane_permute(w, perms[b])
                m = jnp.bitwise_and(lax.shift_right_logical(wb, shifts), 1)
                mf = m.astype(jnp.float32)
                diff = d - e
                acc2 = acc2 + (diff * mf) * diff
                cnt2 = cnt2 + m
            return acc2, cnt2

        return lax.fori_loop(0, QG, group_body, (acc, cnt))

    msk_copy(0, 0).start()
    for c in f32_copies(0, 0):
        c.start()
    for c in f32_copies(1, 1):
        c.start()

    acc = jnp.zeros((16,), jnp.float32)
    cnt = jnp.zeros((16,), jnp.int32)
    for s in range(SLABS):
        ms = s % 2
        msk_copy(s, ms).wait()
        if s + 1 < SLABS:
            msk_copy(s + 1, (s + 1) % 2).start()
        for j in range(FCH):
            k = s * FCH + j
            fs = k % 2
            for c in f32_copies(k, fs):
                c.wait()
            acc, cnt = compute_chunk(fs, j, acc, cnt, ms)
            if k + 2 < NCH:
                for c in f32_copies(k + 2, fs):
                    c.start()

    st_f[...] = acc
    st_c[...] = cnt
    pltpu.sync_copy(st_f, sums_hbm.at[wid])
    pltpu.sync_copy(st_c, cnts_hbm.at[wid])


def _sc_partial(out2d, tgt2d, msk2d_u8):
    mesh = plsc.VectorSubcoreMesh(core_axis_name="c", subcore_axis_name="s")
    return pl.kernel(
        _sc_body,
        mesh=mesh,
        compiler_params=pltpu.CompilerParams(needs_layout_passes=False),
        out_type=[
            jax.ShapeDtypeStruct((NW, L), jnp.float32),
            jax.ShapeDtypeStruct((NW, L), jnp.int32),
        ],
        scratch_types=[
            pltpu.VMEM((8, COLS), jnp.float32),
            pltpu.VMEM((8, COLS), jnp.float32),
            pltpu.VMEM((8, COLS), jnp.float32),
            pltpu.VMEM((8, COLS), jnp.float32),
            pltpu.VMEM((32, COLS), jnp.uint8),
            pltpu.VMEM((32, COLS), jnp.uint8),
            pltpu.VMEM((L,), jnp.float32),
            pltpu.VMEM((L,), jnp.int32),
            pltpu.SemaphoreType.DMA((2,)),
            pltpu.SemaphoreType.DMA((2,)),
        ],
    )(out2d, tgt2d, msk2d_u8)


@jax.jit
def kernel(output, target, mask):
    out2d = output.reshape(ROWS, COLS)
    tgt2d = target.reshape(ROWS, COLS)
    msk2d = mask.reshape(ROWS, COLS)
    msk2d_u8 = msk2d.view(jnp.uint8)

    ssum, scnt = _sc_partial(out2d, tgt2d, msk2d_u8)
    tsum, tcnt = _tc_partial(out2d, tgt2d, msk2d)

    total = jnp.sum(tsum) + jnp.sum(ssum)
    count = jnp.sum(tcnt) + jnp.sum(scnt).astype(jnp.float32)
    return total / count
